# Initial kernel scaffold; baseline (speedup 1.0000x reference)
import jax
import jax.numpy as jnp
from jax.experimental import pallas as pl


def kernel(x, feat_units, label_units):
    raise NotImplementedError("write your pallas kernel here")


# speedup vs baseline: 1.1610x; 1.1610x over previous
import functools

import jax
import jax.numpy as jnp
from jax.experimental import pallas as pl


def _body(x_ref, ft_ref, fl_ref, out_ref, score_ref, *, c):
    xt = x_ref[0]
    ssq = jnp.sum(xt * xt, axis=0, keepdims=True)
    rinv = 1.0 / jnp.maximum(jnp.sqrt(ssq), 1e-12)
    xn_t = xt * rinv
    ft = ft_ref[...]
    csq = jnp.sum(ft * ft, axis=0, keepdims=True)
    cinv = 1.0 / jnp.maximum(jnp.sqrt(csq), 1e-12)
    mn_t = ft * cinv
    s = jax.lax.dot_general(
        xn_t.astype(jnp.bfloat16), mn_t.astype(jnp.bfloat16),
        dimension_numbers=(((0,), (0,)), ((), ())),
        preferred_element_type=jnp.float32)
    score_ref[...] = s
    m = jnp.max(s, axis=1, keepdims=True)
    p = jnp.exp(s - m)
    dinv = 1.0 / jnp.sum(p, axis=1, keepdims=True)
    oxy = jax.lax.dot_general(
        p.astype(jnp.bfloat16), fl_ref[...].astype(jnp.bfloat16),
        dimension_numbers=(((1,), (0,)), ((), ())),
        preferred_element_type=jnp.float32)
    oxy = oxy * dinv
    out_ref[0, :c, :] = xt
    out_ref[0, c:, :] = oxy.T


def kernel(x, feat_units, label_units):
    b, c, h, w = x.shape
    k, ydim = label_units.shape[0], label_units.shape[1]
    n_per_b = h * w
    nb = 512 if n_per_b % 512 == 0 else n_per_b
    jblocks = n_per_b // nb

    x3 = x.reshape(b, c, n_per_b)
    ft = feat_units.T
    fl = jnp.concatenate([feat_units, label_units], axis=1)

    out3, score = pl.pallas_call(
        functools.partial(_body, c=c),
        grid=(b, jblocks),
        in_specs=[
            pl.BlockSpec((1, c, nb), lambda i, j: (i, 0, j)),
            pl.BlockSpec((c, k), lambda i, j: (0, 0)),
            pl.BlockSpec((k, c + ydim), lambda i, j: (0, 0)),
        ],
        out_specs=[
            pl.BlockSpec((1, 2 * c + ydim, nb), lambda i, j: (i, 0, j)),
            pl.BlockSpec((nb, k), lambda i, j, _jb=jblocks: (i * _jb + j, 0)),
        ],
        out_shape=[
            jax.ShapeDtypeStruct((b, 2 * c + ydim, n_per_b), jnp.float32),
            jax.ShapeDtypeStruct((b * n_per_b, k), jnp.float32),
        ],
    )(x3, ft, fl)
    out = out3.reshape(b, 2 * c + ydim, h, w)
    return (out, score)


# speedup vs baseline: 1.1993x; 1.0330x over previous
import functools

import jax
import jax.numpy as jnp
from jax.experimental import pallas as pl
from jax.experimental.pallas import tpu as pltpu


def _body(x_ref, ft_ref, fl_ref, out_ref, score_ref, mn_ref, *, c):
    @pl.when(jnp.logical_and(pl.program_id(0) == 0, pl.program_id(1) == 0))
    def _init():
        ft = ft_ref[...]
        csq = jnp.sum(ft * ft, axis=0, keepdims=True)
        cinv = 1.0 / jnp.maximum(jnp.sqrt(csq), 1e-12)
        mn_ref[...] = (ft * cinv).astype(jnp.bfloat16)

    xt = x_ref[0]
    ssq = jnp.sum(xt * xt, axis=0, keepdims=True)
    rinv = 1.0 / jnp.maximum(jnp.sqrt(ssq), 1e-12)
    xn_t = xt * rinv
    s = jax.lax.dot_general(
        xn_t.astype(jnp.bfloat16), mn_ref[...],
        dimension_numbers=(((0,), (0,)), ((), ())),
        preferred_element_type=jnp.float32)
    score_ref[...] = s
    p = jnp.exp(s)
    dinv = 1.0 / jnp.sum(p, axis=1, keepdims=True)
    oxy = jax.lax.dot_general(
        p.astype(jnp.bfloat16), fl_ref[...],
        dimension_numbers=(((1,), (0,)), ((), ())),
        preferred_element_type=jnp.float32)
    oxy = oxy * dinv
    out_ref[0, :c, :] = xt
    out_ref[0, c:, :] = oxy.T


def kernel(x, feat_units, label_units):
    b, c, h, w = x.shape
    k, ydim = label_units.shape[0], label_units.shape[1]
    n_per_b = h * w
    nb = 512 if n_per_b % 512 == 0 else n_per_b
    jblocks = n_per_b // nb

    x3 = x.reshape(b, c, n_per_b)
    ft = feat_units.T
    fl = jnp.concatenate([feat_units, label_units],
                         axis=1).astype(jnp.bfloat16)

    out3, score = pl.pallas_call(
        functools.partial(_body, c=c),
        grid=(b, jblocks),
        in_specs=[
            pl.BlockSpec((1, c, nb), lambda i, j: (i, 0, j)),
            pl.BlockSpec((c, k), lambda i, j: (0, 0)),
            pl.BlockSpec((k, c + ydim), lambda i, j: (0, 0)),
        ],
        out_specs=[
            pl.BlockSpec((1, 2 * c + ydim, nb), lambda i, j: (i, 0, j)),
            pl.BlockSpec((nb, k), lambda i, j, _jb=jblocks: (i * _jb + j, 0)),
        ],
        out_shape=[
            jax.ShapeDtypeStruct((b, 2 * c + ydim, n_per_b), jnp.float32),
            jax.ShapeDtypeStruct((b * n_per_b, k), jnp.float32),
        ],
        scratch_shapes=[pltpu.VMEM((c, k), jnp.bfloat16)],
    )(x3, ft, fl)
    out = out3.reshape(b, 2 * c + ydim, h, w)
    return (out, score)
